# Initial kernel scaffold; baseline (speedup 1.0000x reference)
import jax
import jax.numpy as jnp
from jax.experimental import pallas as pl


def kernel(x, edge_idx, edge_attr, Wfc, bfc, W1, b1, W2, b2):
    raise NotImplementedError("write your pallas kernel here")


# speedup vs baseline: 7.8294x; 7.8294x over previous
import functools

import jax
import jax.numpy as jnp
from jax import lax
from jax.experimental import pallas as pl
from jax.experimental.pallas import tpu as pltpu
from jax.experimental.pallas import tpu_sc as plsc

NC = 2
NS = 16
NW = NC * NS
CH = 128


def _mesh():
    return plsc.VectorSubcoreMesh(core_axis_name="c", subcore_axis_name="s")


def _zero_rows(rows_v, nrow, d):
    def body(i, carry):
        for t in range(d // 16):
            rows_v[i, pl.ds(16 * t, 16)] = jnp.zeros((16,), jnp.float32)
        return carry
    lax.fori_loop(0, nrow, body, 0)


def _chunked_rows_copy(n, s, copy_one):
    fullch = n // CH
    rem = n - fullch * CH
    tmax = -(-fullch // NS)
    for t in range(tmax):
        k = s + NS * t

        @pl.when(k < fullch)
        def _():
            copy_one(CH * k, CH)
    if rem:
        @pl.when(s == 0)
        def _():
            copy_one(fullch * CH, rem)


def _make_sc_deg(nchunk, degn):
    @functools.partial(
        pl.kernel,
        out_type=jax.ShapeDtypeStruct((NC, degn), jnp.float32),
        mesh=_mesh(),
        scratch_types=[
            pltpu.VMEM((nchunk, CH), jnp.int32),
            pltpu.VMEM((nchunk, CH), jnp.float32),
            pltpu.VMEM((degn // NS,), jnp.float32),
            pltpu.VMEM_SHARED((degn,), jnp.float32),
        ],
        compiler_params=pltpu.CompilerParams(use_tc_tiling_on_sc=False),
    )
    def deg_kernel(dst_e, ew_e, out, dst_v, ew_v, zbuf, deg_sp):
        c = lax.axis_index("c")
        s = lax.axis_index("s")
        w = c * NS + s
        stripe = degn // NS

        def zb(i, carry):
            zbuf[pl.ds(16 * i, 16)] = jnp.zeros((16,), jnp.float32)
            return carry
        lax.fori_loop(0, stripe // 16, zb, 0)
        pltpu.sync_copy(zbuf, deg_sp.at[pl.ds(s * stripe, stripe)])
        plsc.subcore_barrier()

        pltpu.sync_copy(dst_e.at[w], dst_v)
        pltpu.sync_copy(ew_e.at[w], ew_v)

        def body(j, carry):
            pltpu.sync_copy(ew_v.at[j], deg_sp.at[dst_v.at[j]], add=True)
            return carry
        lax.fori_loop(0, nchunk, body, 0)

        plsc.subcore_barrier()
        pltpu.sync_copy(deg_sp.at[pl.ds(s * stripe, stripe)],
                        out.at[c, pl.ds(s * stripe, stripe)])

    return deg_kernel


def _make_sc_conv(n, d, nchunk, split_dims):
    @functools.partial(
        pl.kernel,
        out_type=jax.ShapeDtypeStruct((NC, n, d), jnp.float32),
        mesh=_mesh(),
        scratch_types=[
            pltpu.VMEM((nchunk, CH), jnp.int32),
            pltpu.VMEM((nchunk, CH), jnp.int32),
            pltpu.VMEM((CH, 16), jnp.float32),
            pltpu.VMEM((CH, d), jnp.float32),
            pltpu.VMEM_SHARED((n, d), jnp.float32),
            pltpu.SemaphoreType.DMA,
        ],
        compiler_params=pltpu.CompilerParams(use_tc_tiling_on_sc=False),
    )
    def conv_kernel(table, src_e, dst_e, ew16_e, out,
                    src_v, dst_v, ew16_v, rows_v, acc, sem):
        c = lax.axis_index("c")
        s = lax.axis_index("s")
        w = s if split_dims else c * NS + s
        tbl = table.at[c] if split_dims else table

        _zero_rows(rows_v, CH, d)

        def zero_copy(off, nr):
            pltpu.sync_copy(rows_v.at[pl.ds(0, nr)], acc.at[pl.ds(off, nr)])
        _chunked_rows_copy(n, s, zero_copy)
        plsc.subcore_barrier()

        pltpu.sync_copy(src_e.at[w], src_v)
        pltpu.sync_copy(dst_e.at[w], dst_v)

        def body(j, carry):
            pltpu.sync_copy(ew16_e.at[w, j], ew16_v)
            pltpu.async_copy(tbl.at[src_v.at[j]], rows_v, sem).wait()

            def scale(r, carry2):
                ewb = ew16_v[r]
                for t in range(d // 16):
                    rows_v[r, pl.ds(16 * t, 16)] = (
                        rows_v[r, pl.ds(16 * t, 16)] * ewb)
                return carry2
            lax.fori_loop(0, CH, scale, 0)

            pltpu.sync_copy(rows_v, acc.at[dst_v.at[j]], add=True)
            return carry
        lax.fori_loop(0, nchunk, body, 0)

        plsc.subcore_barrier()

        def out_copy(off, nr):
            pltpu.sync_copy(acc.at[pl.ds(off, nr)], out.at[c, pl.ds(off, nr)])
        _chunked_rows_copy(n, s, out_copy)

    return conv_kernel


def _tc_k1(x, wfc, bfc2, w1, degpair, n, blk):
    din, h1 = wfc.shape
    h2 = w1.shape[1]

    def body(x_ref, wfc_ref, bfc_ref, w1_ref, deg_ref, hp_ref, dinv_ref):
        h0 = jnp.maximum(
            jnp.dot(x_ref[...], wfc_ref[...],
                    preferred_element_type=jnp.float32) + bfc_ref[...], 0.0)
        p1 = jnp.dot(h0, w1_ref[...], preferred_element_type=jnp.float32)
        deg = deg_ref[0, 0] + deg_ref[0, 1] + 1.0
        dinv = jnp.where(deg > 0, lax.rsqrt(jnp.maximum(deg, 1e-12)), 0.0)
        dinv_ref[...] = dinv[:, None]
        hp = p1 * dinv[:, None]
        half = h2 // 2
        dp1 = ((half + 15) // 16) * 16
        pad = jnp.zeros((blk, dp1 - half), jnp.float32)
        hp_ref[0] = jnp.concatenate([hp[:, :half], pad], axis=1)
        hp_ref[1] = jnp.concatenate([hp[:, half:], pad], axis=1)

    dp1 = ((h2 // 2 + 15) // 16) * 16
    return pl.pallas_call(
        body,
        grid=(n // blk,),
        in_specs=[
            pl.BlockSpec((blk, din), lambda i: (i, 0)),
            pl.BlockSpec((din, h1), lambda i: (0, 0)),
            pl.BlockSpec((1, h1), lambda i: (0, 0)),
            pl.BlockSpec((h1, h2), lambda i: (0, 0)),
            pl.BlockSpec((1, NC, blk), lambda i: (i, 0, 0)),
        ],
        out_specs=[
            pl.BlockSpec((NC, blk, dp1), lambda i: (0, i, 0)),
            pl.BlockSpec((blk, 1), lambda i: (i, 0)),
        ],
        out_shape=[
            jax.ShapeDtypeStruct((NC, n, dp1), jnp.float32),
            jax.ShapeDtypeStruct((n, 1), jnp.float32),
        ],
    )(x, wfc, bfc2, w1, degpair)


def _tc_k2(acc1, hp1, dinv, b1r, w2, n, blk):
    h2, dout = w2.shape

    def body(acc_ref, hp1_ref, dinv_ref, b1_ref, w2_ref, hp2_ref):
        half = h2 // 2
        lo = (acc_ref[0] + hp1_ref[0])[:, :half]
        hi = (acc_ref[1] + hp1_ref[1])[:, :half]
        ssum = jnp.concatenate([lo, hi], axis=1)
        h1 = jnp.maximum(ssum * dinv_ref[...] + b1_ref[...], 0.0)
        p2 = jnp.dot(h1, w2_ref[...], preferred_element_type=jnp.float32)
        hp2 = p2 * dinv_ref[...]
        hp2_ref[0] = hp2[:, :dout // 2]
        hp2_ref[1] = hp2[:, dout // 2:]

    dp1 = acc1.shape[2]
    return pl.pallas_call(
        body,
        grid=(n // blk,),
        in_specs=[
            pl.BlockSpec((NC, blk, dp1), lambda i: (0, i, 0)),
            pl.BlockSpec((NC, blk, dp1), lambda i: (0, i, 0)),
            pl.BlockSpec((blk, 1), lambda i: (i, 0)),
            pl.BlockSpec((1, h2), lambda i: (0, 0)),
            pl.BlockSpec((h2, dout), lambda i: (0, 0)),
        ],
        out_specs=pl.BlockSpec((NC, blk, dout // 2), lambda i: (0, i, 0)),
        out_shape=jax.ShapeDtypeStruct((NC, n, dout // 2), jnp.float32),
    )(acc1, hp1, dinv, b1r, w2)


def _tc_k3(acc2, hp2, dinv, b2r, n, blk):
    dout = b2r.shape[1]

    def body(acc_ref, hp2_ref, dinv_ref, b2_ref, out_ref):
        ssum = jnp.concatenate(
            [acc_ref[0] + hp2_ref[0], acc_ref[1] + hp2_ref[1]], axis=1)
        out_ref[...] = jnp.maximum(ssum * dinv_ref[...] + b2_ref[...], 0.0)

    return pl.pallas_call(
        body,
        grid=(n // blk,),
        in_specs=[
            pl.BlockSpec((NC, blk, dout // 2), lambda i: (0, i, 0)),
            pl.BlockSpec((NC, blk, dout // 2), lambda i: (0, i, 0)),
            pl.BlockSpec((blk, 1), lambda i: (i, 0)),
            pl.BlockSpec((1, dout), lambda i: (0, 0)),
        ],
        out_specs=pl.BlockSpec((blk, dout), lambda i: (i, 0)),
        out_shape=jax.ShapeDtypeStruct((n, dout), jnp.float32),
    )(acc2, hp2, dinv, b2r)


def kernel(x, edge_idx, edge_attr, Wfc, bfc, W1, b1, W2, b2):
    n, _ = x.shape
    e = edge_attr.shape[0]
    dout = W2.shape[1]
    blk = 2000

    nchunk = -(-e // (NW * CH))
    epad = NW * nchunk * CH
    nchunk1 = -(-e // (NS * CH))
    epad1 = NS * nchunk1 * CH
    degn = -(-n // (NS * 16)) * (NS * 16)

    dst = jnp.pad(edge_idx[1], (0, epad - e)).reshape(NW, nchunk, CH)
    ew = jnp.pad(edge_attr, (0, epad - e)).reshape(NW, nchunk, CH)
    src1 = jnp.pad(edge_idx[0], (0, epad1 - e)).reshape(NS, nchunk1, CH)
    dst1 = jnp.pad(edge_idx[1], (0, epad1 - e)).reshape(NS, nchunk1, CH)
    ew1 = jnp.pad(edge_attr, (0, epad1 - e)).reshape(NS, nchunk1, CH)
    ew16_1 = jnp.broadcast_to(ew1[..., None], (NS, nchunk1, CH, 16))

    degpair = _make_sc_deg(nchunk, degn)(dst, ew)
    degblk = degpair[:, :n].reshape(NC, n // blk, blk).transpose(1, 0, 2)

    hp1, dinv = _tc_k1(x, Wfc, bfc.reshape(1, -1), W1, degblk, n, blk)

    acc1 = _make_sc_conv(n, hp1.shape[2], nchunk1, True)(hp1, src1, dst1, ew16_1)

    hp2 = _tc_k2(acc1, hp1, dinv, b1.reshape(1, -1), W2, n, blk)

    acc2 = _make_sc_conv(n, dout // 2, nchunk1, True)(hp2, src1, dst1, ew16_1)

    return _tc_k3(acc2, hp2, dinv, b2.reshape(1, -1), n, blk)


# speedup vs baseline: 10.2969x; 1.3152x over previous
import functools

import jax
import jax.numpy as jnp
from jax import lax
from jax.experimental import pallas as pl
from jax.experimental.pallas import tpu as pltpu
from jax.experimental.pallas import tpu_sc as plsc

NC = 2
NS = 16
NW = NC * NS
CH = 128


def _mesh():
    return plsc.VectorSubcoreMesh(core_axis_name="c", subcore_axis_name="s")


def _zero_rows(rows_v, nrow, d):
    def body(i, carry):
        for t in range(d // 16):
            rows_v[i, pl.ds(16 * t, 16)] = jnp.zeros((16,), jnp.float32)
        return carry
    lax.fori_loop(0, nrow, body, 0)


def _chunked_rows_copy(n, s, copy_one):
    fullch = n // CH
    rem = n - fullch * CH
    tmax = -(-fullch // NS)
    for t in range(tmax):
        k = s + NS * t

        @pl.when(k < fullch)
        def _():
            copy_one(CH * k, CH)
    if rem:
        @pl.when(s == 0)
        def _():
            copy_one(fullch * CH, rem)


def _make_sc_deg(nchunk, degn):
    @functools.partial(
        pl.kernel,
        out_type=jax.ShapeDtypeStruct((NC, degn), jnp.float32),
        mesh=_mesh(),
        scratch_types=[
            pltpu.VMEM((nchunk, CH), jnp.int32),
            pltpu.VMEM((nchunk, CH), jnp.float32),
            pltpu.VMEM((degn // NS,), jnp.float32),
            pltpu.VMEM_SHARED((degn,), jnp.float32),
        ],
        compiler_params=pltpu.CompilerParams(use_tc_tiling_on_sc=False),
    )
    def deg_kernel(dst_e, ew_e, out, dst_v, ew_v, zbuf, deg_sp):
        c = lax.axis_index("c")
        s = lax.axis_index("s")
        w = c * NS + s
        stripe = degn // NS

        def zb(i, carry):
            zbuf[pl.ds(16 * i, 16)] = jnp.zeros((16,), jnp.float32)
            return carry
        lax.fori_loop(0, stripe // 16, zb, 0)
        pltpu.sync_copy(zbuf, deg_sp.at[pl.ds(s * stripe, stripe)])
        plsc.subcore_barrier()

        pltpu.sync_copy(dst_e.at[w], dst_v)
        pltpu.sync_copy(ew_e.at[w], ew_v)

        def body(j, carry):
            pltpu.sync_copy(ew_v.at[j], deg_sp.at[dst_v.at[j]], add=True)
            return carry
        lax.fori_loop(0, nchunk, body, 0)

        plsc.subcore_barrier()
        pltpu.sync_copy(deg_sp.at[pl.ds(s * stripe, stripe)],
                        out.at[c, pl.ds(s * stripe, stripe)])

    return deg_kernel


def _make_sc_conv(n, d, nchunk, planes_per_core):
    P = planes_per_core
    assert nchunk % 2 == 0

    @functools.partial(
        pl.kernel,
        out_type=jax.ShapeDtypeStruct((NC * P, n, d), jnp.float32),
        mesh=_mesh(),
        scratch_types=[
            pltpu.VMEM((nchunk, CH), jnp.int32),
            pltpu.VMEM((nchunk, CH), jnp.int32),
            pltpu.VMEM((CH, 16), jnp.float32),
            pltpu.VMEM((CH, 16), jnp.float32),
            pltpu.VMEM((CH, d), jnp.float32),
            pltpu.VMEM((CH, d), jnp.float32),
            pltpu.VMEM_SHARED((n, d), jnp.float32),
            pltpu.SemaphoreType.DMA,
            pltpu.SemaphoreType.DMA,
            pltpu.SemaphoreType.DMA,
            pltpu.SemaphoreType.DMA,
        ],
        compiler_params=pltpu.CompilerParams(use_tc_tiling_on_sc=False),
    )
    def conv_kernel(table, src_e, dst_e, ew16_e, out,
                    src_v, dst_v, ew0, ew1, rows0, rows1, acc,
                    gs0, gs1, es0, es1):
        c = lax.axis_index("c")
        s = lax.axis_index("s")
        w = s
        rows = (rows0, rows1)
        ews = (ew0, ew1)
        gss = (gs0, gs1)
        ess = (es0, es1)

        pltpu.sync_copy(src_e.at[w], src_v)
        pltpu.sync_copy(dst_e.at[w], dst_v)

        for p in range(P):
            plane = c * P + p

            _zero_rows(rows0, CH, d)

            def zero_copy(off, nr):
                pltpu.sync_copy(rows0.at[pl.ds(0, nr)], acc.at[pl.ds(off, nr)])
            _chunked_rows_copy(n, s, zero_copy)
            plsc.subcore_barrier()

            tbl = table.at[plane]

            def start_gather(b, jj):
                pltpu.async_copy(tbl.at[src_v.at[jj]], rows[b], gss[b])
                pltpu.async_copy(ew16_e.at[w, jj], ews[b], ess[b])

            def wait_gather(b):
                pltpu.make_async_copy(
                    tbl.at[src_v.at[0]], rows[b], gss[b]).wait()
                pltpu.make_async_copy(ew16_e.at[w, 0], ews[b], ess[b]).wait()

            def scale(b):
                def srow(r, carry2):
                    ewb = ews[b][r]
                    for t in range(d // 16):
                        rows[b][r, pl.ds(16 * t, 16)] = (
                            rows[b][r, pl.ds(16 * t, 16)] * ewb)
                    return carry2
                lax.fori_loop(0, CH, srow, 0, unroll=2)

            start_gather(0, 0)

            def step(jj, b):
                @pl.when(jj + 1 < nchunk)
                def _():
                    start_gather(1 - b, jj + 1)

                wait_gather(b)
                scale(b)
                pltpu.sync_copy(rows[b], acc.at[dst_v.at[jj]], add=True)

            def body(m, carry):
                step(2 * m, 0)
                step(2 * m + 1, 1)
                return carry
            lax.fori_loop(0, nchunk // 2, body, 0)

            plsc.subcore_barrier()

            def out_copy(off, nr):
                pltpu.sync_copy(acc.at[pl.ds(off, nr)],
                                out.at[plane, pl.ds(off, nr)])
            _chunked_rows_copy(n, s, out_copy)

    return conv_kernel


def _tc_k1(x, wfc, bfc2, w1, degpair, n, blk):
    din, h1 = wfc.shape
    h2 = w1.shape[1]

    def body(x_ref, wfc_ref, bfc_ref, w1_ref, deg_ref, hp_ref, dinv_ref):
        h0 = jnp.maximum(
            jnp.dot(x_ref[...], wfc_ref[...],
                    preferred_element_type=jnp.float32) + bfc_ref[...], 0.0)
        p1 = jnp.dot(h0, w1_ref[...], preferred_element_type=jnp.float32)
        deg = deg_ref[0, 0] + deg_ref[0, 1] + 1.0
        dinv = jnp.where(deg > 0, lax.rsqrt(jnp.maximum(deg, 1e-12)), 0.0)
        dinv_ref[...] = dinv[:, None]
        hp = jnp.concatenate(
            [p1 * dinv[:, None],
             jnp.zeros((blk, 4 * 64 - h2), jnp.float32)], axis=1)
        for p in range(4):
            hp_ref[p] = hp[:, 64 * p:64 * (p + 1)]

    return pl.pallas_call(
        body,
        grid=(n // blk,),
        in_specs=[
            pl.BlockSpec((blk, din), lambda i: (i, 0)),
            pl.BlockSpec((din, h1), lambda i: (0, 0)),
            pl.BlockSpec((1, h1), lambda i: (0, 0)),
            pl.BlockSpec((h1, h2), lambda i: (0, 0)),
            pl.BlockSpec((1, NC, blk), lambda i: (i, 0, 0)),
        ],
        out_specs=[
            pl.BlockSpec((4, blk, 64), lambda i: (0, i, 0)),
            pl.BlockSpec((blk, 1), lambda i: (i, 0)),
        ],
        out_shape=[
            jax.ShapeDtypeStruct((4, n, 64), jnp.float32),
            jax.ShapeDtypeStruct((n, 1), jnp.float32),
        ],
    )(x, wfc, bfc2, w1, degpair)


def _tc_k2(acc1, hp1, dinv, b1r, w2, n, blk):
    h2, dout = w2.shape

    def body(acc_ref, hp1_ref, dinv_ref, b1_ref, w2_ref, hp2_ref):
        parts = [acc_ref[p] + hp1_ref[p] for p in range(4)]
        parts[3] = parts[3][:, :h2 - 3 * 64]
        ssum = jnp.concatenate(parts, axis=1)
        h1 = jnp.maximum(ssum * dinv_ref[...] + b1_ref[...], 0.0)
        p2 = jnp.dot(h1, w2_ref[...], preferred_element_type=jnp.float32)
        hp2 = p2 * dinv_ref[...]
        hp2_ref[0] = hp2[:, :dout // 2]
        hp2_ref[1] = hp2[:, dout // 2:]

    return pl.pallas_call(
        body,
        grid=(n // blk,),
        in_specs=[
            pl.BlockSpec((4, blk, 64), lambda i: (0, i, 0)),
            pl.BlockSpec((4, blk, 64), lambda i: (0, i, 0)),
            pl.BlockSpec((blk, 1), lambda i: (i, 0)),
            pl.BlockSpec((1, h2), lambda i: (0, 0)),
            pl.BlockSpec((h2, dout), lambda i: (0, 0)),
        ],
        out_specs=pl.BlockSpec((NC, blk, dout // 2), lambda i: (0, i, 0)),
        out_shape=jax.ShapeDtypeStruct((NC, n, dout // 2), jnp.float32),
    )(acc1, hp1, dinv, b1r, w2)


def _tc_k3(acc2, hp2, dinv, b2r, n, blk):
    dout = b2r.shape[1]

    def body(acc_ref, hp2_ref, dinv_ref, b2_ref, out_ref):
        ssum = jnp.concatenate(
            [acc_ref[0] + hp2_ref[0], acc_ref[1] + hp2_ref[1]], axis=1)
        out_ref[...] = jnp.maximum(ssum * dinv_ref[...] + b2_ref[...], 0.0)

    return pl.pallas_call(
        body,
        grid=(n // blk,),
        in_specs=[
            pl.BlockSpec((NC, blk, dout // 2), lambda i: (0, i, 0)),
            pl.BlockSpec((NC, blk, dout // 2), lambda i: (0, i, 0)),
            pl.BlockSpec((blk, 1), lambda i: (i, 0)),
            pl.BlockSpec((1, dout), lambda i: (0, 0)),
        ],
        out_specs=pl.BlockSpec((blk, dout), lambda i: (i, 0)),
        out_shape=jax.ShapeDtypeStruct((n, dout), jnp.float32),
    )(acc2, hp2, dinv, b2r)


def kernel(x, edge_idx, edge_attr, Wfc, bfc, W1, b1, W2, b2):
    n, _ = x.shape
    e = edge_attr.shape[0]
    dout = W2.shape[1]
    blk = 2000

    nchunk = -(-e // (NW * CH))
    epad = NW * nchunk * CH
    nchunk1 = -(-e // (NS * CH))
    nchunk1 += nchunk1 % 2
    epad1 = NS * nchunk1 * CH
    degn = -(-n // (NS * 16)) * (NS * 16)

    dst = jnp.pad(edge_idx[1], (0, epad - e)).reshape(NW, nchunk, CH)
    ew = jnp.pad(edge_attr, (0, epad - e)).reshape(NW, nchunk, CH)
    src1 = jnp.pad(edge_idx[0], (0, epad1 - e)).reshape(NS, nchunk1, CH)
    dst1 = jnp.pad(edge_idx[1], (0, epad1 - e)).reshape(NS, nchunk1, CH)
    ew1 = jnp.pad(edge_attr, (0, epad1 - e)).reshape(NS, nchunk1, CH)
    ew16_1 = jnp.broadcast_to(ew1[..., None], (NS, nchunk1, CH, 16))

    degpair = _make_sc_deg(nchunk, degn)(dst, ew)
    degblk = degpair[:, :n].reshape(NC, n // blk, blk).transpose(1, 0, 2)

    hp1, dinv = _tc_k1(x, Wfc, bfc.reshape(1, -1), W1, degblk, n, blk)

    acc1 = _make_sc_conv(n, 64, nchunk1, 2)(hp1, src1, dst1, ew16_1)

    hp2 = _tc_k2(acc1, hp1, dinv, b1.reshape(1, -1), W2, n, blk)

    acc2 = _make_sc_conv(n, dout // 2, nchunk1, 1)(hp2, src1, dst1, ew16_1)

    return _tc_k3(acc2, hp2, dinv, b2.reshape(1, -1), n, blk)


# speedup vs baseline: 11.4358x; 1.1106x over previous
import functools

import jax
import jax.numpy as jnp
from jax import lax
from jax.experimental import pallas as pl
from jax.experimental.pallas import tpu as pltpu
from jax.experimental.pallas import tpu_sc as plsc

NC = 2
NS = 16
NW = NC * NS
CH = 128


def _mesh():
    return plsc.VectorSubcoreMesh(core_axis_name="c", subcore_axis_name="s")


def _zero_rows(rows_v, nrow, d):
    def body(i, carry):
        for t in range(d // 16):
            rows_v[i, pl.ds(16 * t, 16)] = jnp.zeros((16,), jnp.float32)
        return carry
    lax.fori_loop(0, nrow, body, 0)


def _chunked_rows_copy(n, s, copy_one):
    fullch = n // CH
    rem = n - fullch * CH
    tmax = -(-fullch // NS)
    for t in range(tmax):
        k = s + NS * t

        @pl.when(k < fullch)
        def _():
            copy_one(CH * k, CH)
    if rem:
        @pl.when(s == 0)
        def _():
            copy_one(fullch * CH, rem)


def _make_sc_deg(nchunk, degn):
    @functools.partial(
        pl.kernel,
        out_type=jax.ShapeDtypeStruct((NC, degn), jnp.float32),
        mesh=_mesh(),
        scratch_types=[
            pltpu.VMEM((nchunk, CH), jnp.int32),
            pltpu.VMEM((nchunk, CH), jnp.float32),
            pltpu.VMEM((degn // NS,), jnp.float32),
            pltpu.VMEM_SHARED((degn,), jnp.float32),
        ],
        compiler_params=pltpu.CompilerParams(use_tc_tiling_on_sc=False),
    )
    def deg_kernel(dst_e, ew_e, out, dst_v, ew_v, zbuf, deg_sp):
        c = lax.axis_index("c")
        s = lax.axis_index("s")
        w = c * NS + s
        stripe = degn // NS

        def zb(i, carry):
            zbuf[pl.ds(16 * i, 16)] = jnp.zeros((16,), jnp.float32)
            return carry
        lax.fori_loop(0, stripe // 16, zb, 0)
        pltpu.sync_copy(zbuf, deg_sp.at[pl.ds(s * stripe, stripe)])
        plsc.subcore_barrier()

        pltpu.sync_copy(dst_e.at[w], dst_v)
        pltpu.sync_copy(ew_e.at[w], ew_v)

        def body(j, carry):
            pltpu.sync_copy(ew_v.at[j], deg_sp.at[dst_v.at[j]], add=True)
            return carry
        lax.fori_loop(0, nchunk, body, 0)

        plsc.subcore_barrier()
        pltpu.sync_copy(deg_sp.at[pl.ds(s * stripe, stripe)],
                        out.at[c, pl.ds(s * stripe, stripe)])

    return deg_kernel


def _make_sc_conv(n, d, nchunk, planes_per_core):
    P = planes_per_core
    assert nchunk % 2 == 0

    @functools.partial(
        pl.kernel,
        out_type=jax.ShapeDtypeStruct((NC * P, n, d), jnp.float32),
        mesh=_mesh(),
        scratch_types=[
            pltpu.VMEM((nchunk, CH), jnp.int32),
            pltpu.VMEM((nchunk, CH), jnp.int32),
            pltpu.VMEM((CH, 16), jnp.float32),
            pltpu.VMEM((CH, 16), jnp.float32),
            pltpu.VMEM((CH, d), jnp.float32),
            pltpu.VMEM((CH, d), jnp.float32),
            pltpu.VMEM_SHARED((n, d), jnp.float32),
            pltpu.SemaphoreType.DMA,
            pltpu.SemaphoreType.DMA,
            pltpu.SemaphoreType.DMA,
            pltpu.SemaphoreType.DMA,
            pltpu.SemaphoreType.DMA,
            pltpu.SemaphoreType.DMA,
        ],
        compiler_params=pltpu.CompilerParams(use_tc_tiling_on_sc=False),
    )
    def conv_kernel(table, src_e, dst_e, ew16_e, out,
                    src_v, dst_v, ew0, ew1, rows0, rows1, acc,
                    gs0, gs1, es0, es1, ss0, ss1):
        c = lax.axis_index("c")
        s = lax.axis_index("s")
        w = s
        rows = (rows0, rows1)
        ews = (ew0, ew1)
        gss = (gs0, gs1)
        ess = (es0, es1)
        sss = (ss0, ss1)

        pltpu.sync_copy(src_e.at[w], src_v)
        pltpu.sync_copy(dst_e.at[w], dst_v)

        for p in range(P):
            plane = c * P + p

            _zero_rows(rows0, CH, d)

            def zero_copy(off, nr):
                pltpu.sync_copy(rows0.at[pl.ds(0, nr)], acc.at[pl.ds(off, nr)])
            _chunked_rows_copy(n, s, zero_copy)
            plsc.subcore_barrier()

            tbl = table.at[plane]

            def start_gather(b, jj):
                pltpu.async_copy(tbl.at[src_v.at[jj]], rows[b], gss[b])
                pltpu.async_copy(ew16_e.at[w, jj], ews[b], ess[b])

            def wait_gather(b):
                pltpu.make_async_copy(
                    tbl.at[src_v.at[0]], rows[b], gss[b]).wait()
                pltpu.make_async_copy(ew16_e.at[w, 0], ews[b], ess[b]).wait()

            def scale(b):
                @plsc.parallel_loop(0, CH, unroll=4)
                def _(r):
                    ewb = ews[b][r]
                    for t in range(d // 16):
                        rows[b][r, pl.ds(16 * t, 16)] = (
                            rows[b][r, pl.ds(16 * t, 16)] * ewb)

            def start_scatter(b, jj):
                pltpu.async_copy(rows[b], acc.at[dst_v.at[jj]], sss[b],
                                 add=True)

            def wait_scatter(b):
                pltpu.make_async_copy(
                    rows[b], acc.at[dst_v.at[0]], sss[b]).wait()

            start_gather(0, 0)

            def step(jj, b):
                @pl.when(jj + 1 < nchunk)
                def _():
                    @pl.when(jj >= 1)
                    def _():
                        wait_scatter(1 - b)
                    start_gather(1 - b, jj + 1)

                wait_gather(b)
                scale(b)
                start_scatter(b, jj)

            def body(m, carry):
                step(2 * m, 0)
                step(2 * m + 1, 1)
                return carry
            lax.fori_loop(0, nchunk // 2, body, 0)
            wait_scatter(0)
            wait_scatter(1)

            plsc.subcore_barrier()

            def out_copy(off, nr):
                pltpu.sync_copy(acc.at[pl.ds(off, nr)],
                                out.at[plane, pl.ds(off, nr)])
            _chunked_rows_copy(n, s, out_copy)

    return conv_kernel


def _tc_k1(x, wfc, bfc2, w1, degpair, n, blk):
    din, h1 = wfc.shape
    h2 = w1.shape[1]

    def body(x_ref, wfc_ref, bfc_ref, w1_ref, deg_ref, hp_ref, dinv_ref):
        h0 = jnp.maximum(
            jnp.dot(x_ref[...], wfc_ref[...],
                    preferred_element_type=jnp.float32) + bfc_ref[...], 0.0)
        p1 = jnp.dot(h0, w1_ref[...], preferred_element_type=jnp.float32)
        deg = deg_ref[0, 0] + deg_ref[0, 1] + 1.0
        dinv = jnp.where(deg > 0, lax.rsqrt(jnp.maximum(deg, 1e-12)), 0.0)
        dinv_ref[...] = dinv[:, None]
        hp = jnp.concatenate(
            [p1 * dinv[:, None],
             jnp.zeros((blk, 4 * 64 - h2), jnp.float32)], axis=1)
        for p in range(4):
            hp_ref[p] = hp[:, 64 * p:64 * (p + 1)]

    return pl.pallas_call(
        body,
        grid=(n // blk,),
        in_specs=[
            pl.BlockSpec((blk, din), lambda i: (i, 0)),
            pl.BlockSpec((din, h1), lambda i: (0, 0)),
            pl.BlockSpec((1, h1), lambda i: (0, 0)),
            pl.BlockSpec((h1, h2), lambda i: (0, 0)),
            pl.BlockSpec((1, NC, blk), lambda i: (i, 0, 0)),
        ],
        out_specs=[
            pl.BlockSpec((4, blk, 64), lambda i: (0, i, 0)),
            pl.BlockSpec((blk, 1), lambda i: (i, 0)),
        ],
        out_shape=[
            jax.ShapeDtypeStruct((4, n, 64), jnp.float32),
            jax.ShapeDtypeStruct((n, 1), jnp.float32),
        ],
    )(x, wfc, bfc2, w1, degpair)


def _tc_k2(acc1, hp1, dinv, b1r, w2, n, blk):
    h2, dout = w2.shape

    def body(acc_ref, hp1_ref, dinv_ref, b1_ref, w2_ref, hp2_ref):
        parts = [acc_ref[p] + hp1_ref[p] for p in range(4)]
        parts[3] = parts[3][:, :h2 - 3 * 64]
        ssum = jnp.concatenate(parts, axis=1)
        h1 = jnp.maximum(ssum * dinv_ref[...] + b1_ref[...], 0.0)
        p2 = jnp.dot(h1, w2_ref[...], preferred_element_type=jnp.float32)
        hp2 = p2 * dinv_ref[...]
        hp2_ref[0] = hp2[:, :dout // 2]
        hp2_ref[1] = hp2[:, dout // 2:]

    return pl.pallas_call(
        body,
        grid=(n // blk,),
        in_specs=[
            pl.BlockSpec((4, blk, 64), lambda i: (0, i, 0)),
            pl.BlockSpec((4, blk, 64), lambda i: (0, i, 0)),
            pl.BlockSpec((blk, 1), lambda i: (i, 0)),
            pl.BlockSpec((1, h2), lambda i: (0, 0)),
            pl.BlockSpec((h2, dout), lambda i: (0, 0)),
        ],
        out_specs=pl.BlockSpec((NC, blk, dout // 2), lambda i: (0, i, 0)),
        out_shape=jax.ShapeDtypeStruct((NC, n, dout // 2), jnp.float32),
    )(acc1, hp1, dinv, b1r, w2)


def _tc_k3(acc2, hp2, dinv, b2r, n, blk):
    dout = b2r.shape[1]

    def body(acc_ref, hp2_ref, dinv_ref, b2_ref, out_ref):
        ssum = jnp.concatenate(
            [acc_ref[0] + hp2_ref[0], acc_ref[1] + hp2_ref[1]], axis=1)
        out_ref[...] = jnp.maximum(ssum * dinv_ref[...] + b2_ref[...], 0.0)

    return pl.pallas_call(
        body,
        grid=(n // blk,),
        in_specs=[
            pl.BlockSpec((NC, blk, dout // 2), lambda i: (0, i, 0)),
            pl.BlockSpec((NC, blk, dout // 2), lambda i: (0, i, 0)),
            pl.BlockSpec((blk, 1), lambda i: (i, 0)),
            pl.BlockSpec((1, dout), lambda i: (0, 0)),
        ],
        out_specs=pl.BlockSpec((blk, dout), lambda i: (i, 0)),
        out_shape=jax.ShapeDtypeStruct((n, dout), jnp.float32),
    )(acc2, hp2, dinv, b2r)


def kernel(x, edge_idx, edge_attr, Wfc, bfc, W1, b1, W2, b2):
    n, _ = x.shape
    e = edge_attr.shape[0]
    dout = W2.shape[1]
    blk = 2000

    nchunk = -(-e // (NW * CH))
    epad = NW * nchunk * CH
    nchunk1 = -(-e // (NS * CH))
    nchunk1 += nchunk1 % 2
    epad1 = NS * nchunk1 * CH
    degn = -(-n // (NS * 16)) * (NS * 16)

    dst = jnp.pad(edge_idx[1], (0, epad - e)).reshape(NW, nchunk, CH)
    ew = jnp.pad(edge_attr, (0, epad - e)).reshape(NW, nchunk, CH)
    src1 = jnp.pad(edge_idx[0], (0, epad1 - e)).reshape(NS, nchunk1, CH)
    dst1 = jnp.pad(edge_idx[1], (0, epad1 - e)).reshape(NS, nchunk1, CH)
    ew1 = jnp.pad(edge_attr, (0, epad1 - e)).reshape(NS, nchunk1, CH)
    ew16_1 = jnp.broadcast_to(ew1[..., None], (NS, nchunk1, CH, 16))

    degpair = _make_sc_deg(nchunk, degn)(dst, ew)
    degblk = degpair[:, :n].reshape(NC, n // blk, blk).transpose(1, 0, 2)

    hp1, dinv = _tc_k1(x, Wfc, bfc.reshape(1, -1), W1, degblk, n, blk)

    acc1 = _make_sc_conv(n, 64, nchunk1, 2)(hp1, src1, dst1, ew16_1)

    hp2 = _tc_k2(acc1, hp1, dinv, b1.reshape(1, -1), W2, n, blk)

    acc2 = _make_sc_conv(n, dout // 2, nchunk1, 1)(hp2, src1, dst1, ew16_1)

    return _tc_k3(acc2, hp2, dinv, b2.reshape(1, -1), n, blk)


# speedup vs baseline: 11.6294x; 1.0169x over previous
import functools

import jax
import jax.numpy as jnp
from jax import lax
from jax.experimental import pallas as pl
from jax.experimental.pallas import tpu as pltpu
from jax.experimental.pallas import tpu_sc as plsc

NC = 2
NS = 16
NW = NC * NS
CH = 128


def _mesh():
    return plsc.VectorSubcoreMesh(core_axis_name="c", subcore_axis_name="s")


def _zero_rows(rows_v, nrow, d):
    def body(i, carry):
        for t in range(d // 16):
            rows_v[i, pl.ds(16 * t, 16)] = jnp.zeros((16,), jnp.float32)
        return carry
    lax.fori_loop(0, nrow, body, 0)


def _chunked_rows_copy(n, s, copy_one):
    fullch = n // CH
    rem = n - fullch * CH
    tmax = -(-fullch // NS)
    for t in range(tmax):
        k = s + NS * t

        @pl.when(k < fullch)
        def _():
            copy_one(CH * k, CH)
    if rem:
        @pl.when(s == 0)
        def _():
            copy_one(fullch * CH, rem)


def _make_sc_deg(nchunk, degn):
    @functools.partial(
        pl.kernel,
        out_type=jax.ShapeDtypeStruct((NC, degn), jnp.float32),
        mesh=_mesh(),
        scratch_types=[
            pltpu.VMEM((nchunk, CH), jnp.int32),
            pltpu.VMEM((nchunk, CH), jnp.float32),
            pltpu.VMEM((degn // NS,), jnp.float32),
            pltpu.VMEM_SHARED((degn,), jnp.float32),
        ],
        compiler_params=pltpu.CompilerParams(use_tc_tiling_on_sc=False),
    )
    def deg_kernel(dst_e, ew_e, out, dst_v, ew_v, zbuf, deg_sp):
        c = lax.axis_index("c")
        s = lax.axis_index("s")
        w = c * NS + s
        stripe = degn // NS

        def zb(i, carry):
            zbuf[pl.ds(16 * i, 16)] = jnp.zeros((16,), jnp.float32)
            return carry
        lax.fori_loop(0, stripe // 16, zb, 0)
        pltpu.sync_copy(zbuf, deg_sp.at[pl.ds(s * stripe, stripe)])
        plsc.subcore_barrier()

        pltpu.sync_copy(dst_e.at[w], dst_v)
        pltpu.sync_copy(ew_e.at[w], ew_v)

        def body(j, carry):
            pltpu.sync_copy(ew_v.at[j], deg_sp.at[dst_v.at[j]], add=True)
            return carry
        lax.fori_loop(0, nchunk, body, 0)

        plsc.subcore_barrier()
        pltpu.sync_copy(deg_sp.at[pl.ds(s * stripe, stripe)],
                        out.at[c, pl.ds(s * stripe, stripe)])

    return deg_kernel


def _make_sc_conv(n, d, nchunk, planes_per_core):
    P = planes_per_core
    NB = 3
    assert nchunk % NB == 0

    @functools.partial(
        pl.kernel,
        out_type=jax.ShapeDtypeStruct((NC * P, n, d), jnp.float32),
        mesh=_mesh(),
        scratch_types=[
            pltpu.VMEM((nchunk, CH), jnp.int32),
            pltpu.VMEM((nchunk, CH), jnp.int32),
            pltpu.VMEM((nchunk, CH), jnp.float32),
            [pltpu.VMEM((CH, d), jnp.float32) for _ in range(NB)],
            pltpu.VMEM_SHARED((n, d), jnp.float32),
            [pltpu.SemaphoreType.DMA for _ in range(NB)],
            [pltpu.SemaphoreType.DMA for _ in range(NB)],
        ],
        compiler_params=pltpu.CompilerParams(use_tc_tiling_on_sc=False),
    )
    def conv_kernel(table, src_e, dst_e, ew_e, out,
                    src_v, dst_v, ew_v, rows, acc, gss, sss):
        c = lax.axis_index("c")
        s = lax.axis_index("s")
        w = s

        pltpu.sync_copy(src_e.at[w], src_v)
        pltpu.sync_copy(dst_e.at[w], dst_v)
        pltpu.sync_copy(ew_e.at[w], ew_v)

        for p in range(P):
            plane = c * P + p

            _zero_rows(rows[0], CH, d)

            def zero_copy(off, nr):
                pltpu.sync_copy(rows[0].at[pl.ds(0, nr)],
                                acc.at[pl.ds(off, nr)])
            _chunked_rows_copy(n, s, zero_copy)
            plsc.subcore_barrier()

            tbl = table.at[plane]

            def start_gather(b, jj):
                pltpu.async_copy(tbl.at[src_v.at[jj]], rows[b], gss[b])

            def wait_gather(b):
                pltpu.make_async_copy(
                    tbl.at[src_v.at[0]], rows[b], gss[b]).wait()

            def scale(b, jj):
                @plsc.parallel_loop(0, CH // 16, unroll=2)
                def _(g):
                    vec = ew_v[jj, pl.ds(16 * g, 16)]
                    for lane in range(16):
                        ewb = jnp.full((16,), vec[lane])
                        r = 16 * g + lane
                        for t in range(d // 16):
                            rows[b][r, pl.ds(16 * t, 16)] = (
                                rows[b][r, pl.ds(16 * t, 16)] * ewb)

            def start_scatter(b, jj):
                pltpu.async_copy(rows[b], acc.at[dst_v.at[jj]], sss[b],
                                 add=True)

            def wait_scatter(b):
                pltpu.make_async_copy(
                    rows[b], acc.at[dst_v.at[0]], sss[b]).wait()

            for b in range(NB - 1):
                start_gather(b, b)

            def step(jj, b):
                nb = (b + NB - 1) % NB

                @pl.when(jj + NB - 1 < nchunk)
                def _():
                    @pl.when(jj >= 1)
                    def _():
                        wait_scatter(nb)
                    start_gather(nb, jj + NB - 1)

                wait_gather(b)
                scale(b, jj)
                start_scatter(b, jj)

            def body(m, carry):
                for b in range(NB):
                    step(NB * m + b, b)
                return carry
            lax.fori_loop(0, nchunk // NB, body, 0)
            for b in range(NB):
                wait_scatter(b)

            plsc.subcore_barrier()

            def out_copy(off, nr):
                pltpu.sync_copy(acc.at[pl.ds(off, nr)],
                                out.at[plane, pl.ds(off, nr)])
            _chunked_rows_copy(n, s, out_copy)

    return conv_kernel


def _tc_k1(x, wfc, bfc2, w1, degpair, n, blk):
    din, h1 = wfc.shape
    h2 = w1.shape[1]

    def body(x_ref, wfc_ref, bfc_ref, w1_ref, deg_ref, hp_ref, dinv_ref):
        h0 = jnp.maximum(
            jnp.dot(x_ref[...], wfc_ref[...],
                    preferred_element_type=jnp.float32) + bfc_ref[...], 0.0)
        p1 = jnp.dot(h0, w1_ref[...], preferred_element_type=jnp.float32)
        deg = deg_ref[0, 0] + deg_ref[0, 1] + 1.0
        dinv = jnp.where(deg > 0, lax.rsqrt(jnp.maximum(deg, 1e-12)), 0.0)
        dinv_ref[...] = dinv[:, None]
        hp = jnp.concatenate(
            [p1 * dinv[:, None],
             jnp.zeros((blk, 4 * 64 - h2), jnp.float32)], axis=1)
        for p in range(4):
            hp_ref[p] = hp[:, 64 * p:64 * (p + 1)]

    return pl.pallas_call(
        body,
        grid=(n // blk,),
        in_specs=[
            pl.BlockSpec((blk, din), lambda i: (i, 0)),
            pl.BlockSpec((din, h1), lambda i: (0, 0)),
            pl.BlockSpec((1, h1), lambda i: (0, 0)),
            pl.BlockSpec((h1, h2), lambda i: (0, 0)),
            pl.BlockSpec((1, NC, blk), lambda i: (i, 0, 0)),
        ],
        out_specs=[
            pl.BlockSpec((4, blk, 64), lambda i: (0, i, 0)),
            pl.BlockSpec((blk, 1), lambda i: (i, 0)),
        ],
        out_shape=[
            jax.ShapeDtypeStruct((4, n, 64), jnp.float32),
            jax.ShapeDtypeStruct((n, 1), jnp.float32),
        ],
    )(x, wfc, bfc2, w1, degpair)


def _tc_k2(acc1, hp1, dinv, b1r, w2, n, blk):
    h2, dout = w2.shape

    def body(acc_ref, hp1_ref, dinv_ref, b1_ref, w2_ref, hp2_ref):
        parts = [acc_ref[p] + hp1_ref[p] for p in range(4)]
        parts[3] = parts[3][:, :h2 - 3 * 64]
        ssum = jnp.concatenate(parts, axis=1)
        h1 = jnp.maximum(ssum * dinv_ref[...] + b1_ref[...], 0.0)
        p2 = jnp.dot(h1, w2_ref[...], preferred_element_type=jnp.float32)
        hp2 = p2 * dinv_ref[...]
        hp2_ref[0] = hp2[:, :dout // 2]
        hp2_ref[1] = hp2[:, dout // 2:]

    return pl.pallas_call(
        body,
        grid=(n // blk,),
        in_specs=[
            pl.BlockSpec((4, blk, 64), lambda i: (0, i, 0)),
            pl.BlockSpec((4, blk, 64), lambda i: (0, i, 0)),
            pl.BlockSpec((blk, 1), lambda i: (i, 0)),
            pl.BlockSpec((1, h2), lambda i: (0, 0)),
            pl.BlockSpec((h2, dout), lambda i: (0, 0)),
        ],
        out_specs=pl.BlockSpec((NC, blk, dout // 2), lambda i: (0, i, 0)),
        out_shape=jax.ShapeDtypeStruct((NC, n, dout // 2), jnp.float32),
    )(acc1, hp1, dinv, b1r, w2)


def _tc_k3(acc2, hp2, dinv, b2r, n, blk):
    dout = b2r.shape[1]

    def body(acc_ref, hp2_ref, dinv_ref, b2_ref, out_ref):
        ssum = jnp.concatenate(
            [acc_ref[0] + hp2_ref[0], acc_ref[1] + hp2_ref[1]], axis=1)
        out_ref[...] = jnp.maximum(ssum * dinv_ref[...] + b2_ref[...], 0.0)

    return pl.pallas_call(
        body,
        grid=(n // blk,),
        in_specs=[
            pl.BlockSpec((NC, blk, dout // 2), lambda i: (0, i, 0)),
            pl.BlockSpec((NC, blk, dout // 2), lambda i: (0, i, 0)),
            pl.BlockSpec((blk, 1), lambda i: (i, 0)),
            pl.BlockSpec((1, dout), lambda i: (0, 0)),
        ],
        out_specs=pl.BlockSpec((blk, dout), lambda i: (i, 0)),
        out_shape=jax.ShapeDtypeStruct((n, dout), jnp.float32),
    )(acc2, hp2, dinv, b2r)


def kernel(x, edge_idx, edge_attr, Wfc, bfc, W1, b1, W2, b2):
    n, _ = x.shape
    e = edge_attr.shape[0]
    dout = W2.shape[1]
    blk = 2000

    nchunk = -(-e // (NW * CH))
    epad = NW * nchunk * CH
    nchunk1 = -(-e // (NS * CH))
    nchunk1 = -(-nchunk1 // 3) * 3
    epad1 = NS * nchunk1 * CH
    degn = -(-n // (NS * 16)) * (NS * 16)

    dst = jnp.pad(edge_idx[1], (0, epad - e)).reshape(NW, nchunk, CH)
    ew = jnp.pad(edge_attr, (0, epad - e)).reshape(NW, nchunk, CH)
    src1 = jnp.pad(edge_idx[0], (0, epad1 - e)).reshape(NS, nchunk1, CH)
    dst1 = jnp.pad(edge_idx[1], (0, epad1 - e)).reshape(NS, nchunk1, CH)
    ew1 = jnp.pad(edge_attr, (0, epad1 - e)).reshape(NS, nchunk1, CH)

    degpair = _make_sc_deg(nchunk, degn)(dst, ew)
    degblk = degpair[:, :n].reshape(NC, n // blk, blk).transpose(1, 0, 2)

    hp1, dinv = _tc_k1(x, Wfc, bfc.reshape(1, -1), W1, degblk, n, blk)

    acc1 = _make_sc_conv(n, 64, nchunk1, 2)(hp1, src1, dst1, ew1)

    hp2 = _tc_k2(acc1, hp1, dinv, b1.reshape(1, -1), W2, n, blk)

    acc2 = _make_sc_conv(n, dout // 2, nchunk1, 1)(hp2, src1, dst1, ew1)

    return _tc_k3(acc2, hp2, dinv, b2.reshape(1, -1), n, blk)


# speedup vs baseline: 12.1772x; 1.0471x over previous
import functools

import jax
import jax.numpy as jnp
from jax import lax
from jax.experimental import pallas as pl
from jax.experimental.pallas import tpu as pltpu
from jax.experimental.pallas import tpu_sc as plsc

NC = 2
NS = 16
NW = NC * NS
CH = 128


def _mesh():
    return plsc.VectorSubcoreMesh(core_axis_name="c", subcore_axis_name="s")


def _zero_rows(rows_v, nrow, d):
    def body(i, carry):
        for t in range(d // 16):
            rows_v[i, pl.ds(16 * t, 16)] = jnp.zeros((16,), jnp.float32)
        return carry
    lax.fori_loop(0, nrow, body, 0)


def _chunked_rows_copy(n, s, copy_one):
    fullch = n // CH
    rem = n - fullch * CH
    tmax = -(-fullch // NS)
    for t in range(tmax):
        k = s + NS * t

        @pl.when(k < fullch)
        def _():
            copy_one(CH * k, CH)
    if rem:
        @pl.when(s == 0)
        def _():
            copy_one(fullch * CH, rem)


def _make_sc_deg(nchunk, degn):
    @functools.partial(
        pl.kernel,
        out_type=jax.ShapeDtypeStruct((NC, degn), jnp.float32),
        mesh=_mesh(),
        scratch_types=[
            pltpu.VMEM((nchunk, CH), jnp.int32),
            pltpu.VMEM((nchunk, CH), jnp.float32),
            pltpu.VMEM((degn // NS,), jnp.float32),
            pltpu.VMEM_SHARED((degn,), jnp.float32),
        ],
        compiler_params=pltpu.CompilerParams(use_tc_tiling_on_sc=False),
    )
    def deg_kernel(dst_e, ew_e, out, dst_v, ew_v, zbuf, deg_sp):
        c = lax.axis_index("c")
        s = lax.axis_index("s")
        w = c * NS + s
        stripe = degn // NS

        def zb(i, carry):
            zbuf[pl.ds(16 * i, 16)] = jnp.zeros((16,), jnp.float32)
            return carry
        lax.fori_loop(0, stripe // 16, zb, 0)
        pltpu.sync_copy(zbuf, deg_sp.at[pl.ds(s * stripe, stripe)])
        plsc.subcore_barrier()

        pltpu.sync_copy(dst_e.at[w], dst_v)
        pltpu.sync_copy(ew_e.at[w], ew_v)

        def body(j, carry):
            pltpu.sync_copy(ew_v.at[j], deg_sp.at[dst_v.at[j]], add=True)
            return carry
        lax.fori_loop(0, nchunk, body, 0)

        plsc.subcore_barrier()
        pltpu.sync_copy(deg_sp.at[pl.ds(s * stripe, stripe)],
                        out.at[c, pl.ds(s * stripe, stripe)])

    return deg_kernel


def _make_sc_conv(n, d, nchunk, planes_per_core):
    P = planes_per_core
    NB = 3
    assert nchunk % NB == 0

    @functools.partial(
        pl.kernel,
        out_type=jax.ShapeDtypeStruct((NC * P, n, d), jnp.float32),
        mesh=_mesh(),
        scratch_types=[
            pltpu.VMEM((nchunk, CH), jnp.int32),
            pltpu.VMEM((nchunk, CH), jnp.int32),
            pltpu.VMEM((nchunk, CH), jnp.float32),
            [pltpu.VMEM((CH, d), jnp.float32) for _ in range(NB)],
            pltpu.VMEM_SHARED((n, d), jnp.float32),
            [pltpu.SemaphoreType.DMA for _ in range(NB)],
            [pltpu.SemaphoreType.DMA for _ in range(NB)],
        ],
        compiler_params=pltpu.CompilerParams(use_tc_tiling_on_sc=False),
    )
    def conv_kernel(table, src_e, dst_e, ew_e, out,
                    src_v, dst_v, ew_v, rows, acc, gss, sss):
        c = lax.axis_index("c")
        s = lax.axis_index("s")
        w = s

        pltpu.sync_copy(src_e.at[w], src_v)
        pltpu.sync_copy(dst_e.at[w], dst_v)
        pltpu.sync_copy(ew_e.at[w], ew_v)

        for p in range(P):
            plane = c * P + p

            _zero_rows(rows[0], CH, d)

            def zero_copy(off, nr):
                pltpu.sync_copy(rows[0].at[pl.ds(0, nr)],
                                acc.at[pl.ds(off, nr)])
            _chunked_rows_copy(n, s, zero_copy)
            plsc.subcore_barrier()

            tbl = table.at[plane]

            def start_gather(b, jj):
                pltpu.async_copy(tbl.at[src_v.at[jj]], rows[b], gss[b])

            def wait_gather(b):
                pltpu.make_async_copy(
                    tbl.at[src_v.at[0]], rows[b], gss[b]).wait()

            def scale(b, jj):
                @plsc.parallel_loop(0, CH // 16, unroll=2)
                def _(g):
                    vec = ew_v[jj, pl.ds(16 * g, 16)]
                    for lane in range(16):
                        ewb = jnp.full((16,), vec[lane])
                        r = 16 * g + lane
                        for t in range(d // 16):
                            rows[b][r, pl.ds(16 * t, 16)] = (
                                rows[b][r, pl.ds(16 * t, 16)] * ewb)

            def start_scatter(b, jj):
                pltpu.async_copy(rows[b], acc.at[dst_v.at[jj]], sss[b],
                                 add=True)

            def wait_scatter(b):
                pltpu.make_async_copy(
                    rows[b], acc.at[dst_v.at[0]], sss[b]).wait()

            for b in range(NB - 1):
                start_gather(b, b)

            def step(jj, b):
                nb = (b + NB - 1) % NB

                @pl.when(jj + NB - 1 < nchunk)
                def _():
                    @pl.when(jj >= 1)
                    def _():
                        wait_scatter(nb)
                    start_gather(nb, jj + NB - 1)

                wait_gather(b)
                scale(b, jj)
                start_scatter(b, jj)

            def body(m, carry):
                for b in range(NB):
                    step(NB * m + b, b)
                return carry
            lax.fori_loop(0, nchunk // NB, body, 0)
            for b in range(NB):
                wait_scatter(b)

            plsc.subcore_barrier()

            def out_copy(off, nr):
                pltpu.sync_copy(acc.at[pl.ds(off, nr)],
                                out.at[plane, pl.ds(off, nr)])
            _chunked_rows_copy(n, s, out_copy)

    return conv_kernel


def _tc_k1a(x, wfc, bfc2, w1, n, blk):
    din, h1 = wfc.shape
    h2 = w1.shape[1]

    def body(x_ref, wfc_ref, bfc_ref, w1_ref, p1_ref):
        h0 = jnp.maximum(
            jnp.dot(x_ref[...], wfc_ref[...],
                    preferred_element_type=jnp.float32) + bfc_ref[...], 0.0)
        p1_ref[...] = jnp.dot(h0, w1_ref[...],
                              preferred_element_type=jnp.float32)

    return pl.pallas_call(
        body,
        grid=(n // blk,),
        in_specs=[
            pl.BlockSpec((blk, din), lambda i: (i, 0)),
            pl.BlockSpec((din, h1), lambda i: (0, 0)),
            pl.BlockSpec((1, h1), lambda i: (0, 0)),
            pl.BlockSpec((h1, h2), lambda i: (0, 0)),
        ],
        out_specs=pl.BlockSpec((blk, h2), lambda i: (i, 0)),
        out_shape=jax.ShapeDtypeStruct((n, h2), jnp.float32),
    )(x, wfc, bfc2, w1)


def _tc_k1b(p1, degpair, n, blk):
    h2 = p1.shape[1]

    def body(p1_ref, deg_ref, hp_ref, dinv_ref):
        deg = deg_ref[0, 0] + deg_ref[0, 1] + 1.0
        dinv = jnp.where(deg > 0, lax.rsqrt(jnp.maximum(deg, 1e-12)), 0.0)
        dinv_ref[...] = dinv[:, None]
        hp = jnp.concatenate(
            [p1_ref[...] * dinv[:, None],
             jnp.zeros((blk, 4 * 64 - h2), jnp.float32)], axis=1)
        for p in range(4):
            hp_ref[p] = hp[:, 64 * p:64 * (p + 1)]

    return pl.pallas_call(
        body,
        grid=(n // blk,),
        in_specs=[
            pl.BlockSpec((blk, h2), lambda i: (i, 0)),
            pl.BlockSpec((1, NC, blk), lambda i: (i, 0, 0)),
        ],
        out_specs=[
            pl.BlockSpec((4, blk, 64), lambda i: (0, i, 0)),
            pl.BlockSpec((blk, 1), lambda i: (i, 0)),
        ],
        out_shape=[
            jax.ShapeDtypeStruct((4, n, 64), jnp.float32),
            jax.ShapeDtypeStruct((n, 1), jnp.float32),
        ],
    )(p1, degpair)


def _tc_k2(acc1, hp1, dinv, b1r, w2, n, blk):
    h2, dout = w2.shape

    def body(acc_ref, hp1_ref, dinv_ref, b1_ref, w2_ref, hp2_ref):
        parts = [acc_ref[p] + hp1_ref[p] for p in range(4)]
        parts[3] = parts[3][:, :h2 - 3 * 64]
        ssum = jnp.concatenate(parts, axis=1)
        h1 = jnp.maximum(ssum * dinv_ref[...] + b1_ref[...], 0.0)
        p2 = jnp.dot(h1, w2_ref[...], preferred_element_type=jnp.float32)
        hp2 = p2 * dinv_ref[...]
        hp2_ref[0] = hp2[:, :dout // 2]
        hp2_ref[1] = hp2[:, dout // 2:]

    return pl.pallas_call(
        body,
        grid=(n // blk,),
        in_specs=[
            pl.BlockSpec((4, blk, 64), lambda i: (0, i, 0)),
            pl.BlockSpec((4, blk, 64), lambda i: (0, i, 0)),
            pl.BlockSpec((blk, 1), lambda i: (i, 0)),
            pl.BlockSpec((1, h2), lambda i: (0, 0)),
            pl.BlockSpec((h2, dout), lambda i: (0, 0)),
        ],
        out_specs=pl.BlockSpec((NC, blk, dout // 2), lambda i: (0, i, 0)),
        out_shape=jax.ShapeDtypeStruct((NC, n, dout // 2), jnp.float32),
    )(acc1, hp1, dinv, b1r, w2)


def _tc_k3(acc2, hp2, dinv, b2r, n, blk):
    dout = b2r.shape[1]

    def body(acc_ref, hp2_ref, dinv_ref, b2_ref, out_ref):
        ssum = jnp.concatenate(
            [acc_ref[0] + hp2_ref[0], acc_ref[1] + hp2_ref[1]], axis=1)
        out_ref[...] = jnp.maximum(ssum * dinv_ref[...] + b2_ref[...], 0.0)

    return pl.pallas_call(
        body,
        grid=(n // blk,),
        in_specs=[
            pl.BlockSpec((NC, blk, dout // 2), lambda i: (0, i, 0)),
            pl.BlockSpec((NC, blk, dout // 2), lambda i: (0, i, 0)),
            pl.BlockSpec((blk, 1), lambda i: (i, 0)),
            pl.BlockSpec((1, dout), lambda i: (0, 0)),
        ],
        out_specs=pl.BlockSpec((blk, dout), lambda i: (i, 0)),
        out_shape=jax.ShapeDtypeStruct((n, dout), jnp.float32),
    )(acc2, hp2, dinv, b2r)


def kernel(x, edge_idx, edge_attr, Wfc, bfc, W1, b1, W2, b2):
    n, _ = x.shape
    e = edge_attr.shape[0]
    dout = W2.shape[1]
    blk = 2000

    nchunk = -(-e // (NW * CH))
    epad = NW * nchunk * CH
    nchunk1 = -(-e // (NS * CH))
    nchunk1 = -(-nchunk1 // 3) * 3
    epad1 = NS * nchunk1 * CH
    degn = -(-n // (NS * 16)) * (NS * 16)

    dst = jnp.pad(edge_idx[1], (0, epad - e)).reshape(NW, nchunk, CH)
    ew = jnp.pad(edge_attr, (0, epad - e)).reshape(NW, nchunk, CH)
    src1 = jnp.pad(edge_idx[0], (0, epad1 - e)).reshape(NS, nchunk1, CH)
    dst1 = jnp.pad(edge_idx[1], (0, epad1 - e)).reshape(NS, nchunk1, CH)
    ew1 = jnp.pad(edge_attr, (0, epad1 - e)).reshape(NS, nchunk1, CH)

    p1 = _tc_k1a(x, Wfc, bfc.reshape(1, -1), W1, n, blk)
    degpair = _make_sc_deg(nchunk, degn)(dst, ew)
    degblk = degpair[:, :n].reshape(NC, n // blk, blk).transpose(1, 0, 2)
    hp1, dinv = _tc_k1b(p1, degblk, n, blk)

    acc1 = _make_sc_conv(n, 64, nchunk1, 2)(hp1, src1, dst1, ew1)

    hp2 = _tc_k2(acc1, hp1, dinv, b1.reshape(1, -1), W2, n, blk)

    acc2 = _make_sc_conv(n, dout // 2, nchunk1, 1)(hp2, src1, dst1, ew1)

    return _tc_k3(acc2, hp2, dinv, b2.reshape(1, -1), n, blk)


# speedup vs baseline: 12.6461x; 1.0385x over previous
import functools

import jax
import jax.numpy as jnp
from jax import lax
from jax.experimental import pallas as pl
from jax.experimental.pallas import tpu as pltpu
from jax.experimental.pallas import tpu_sc as plsc

NC = 2
NS = 16
NW = NC * NS
CH = 128


def _mesh():
    return plsc.VectorSubcoreMesh(core_axis_name="c", subcore_axis_name="s")


def _zero_rows(rows_v, nrow, d):
    def body(i, carry):
        for t in range(d // 16):
            rows_v[i, pl.ds(16 * t, 16)] = jnp.zeros((16,), jnp.float32)
        return carry
    lax.fori_loop(0, nrow, body, 0)


def _chunked_rows_copy(n, s, copy_one):
    fullch = n // CH
    rem = n - fullch * CH
    tmax = -(-fullch // NS)
    for t in range(tmax):
        k = s + NS * t

        @pl.when(k < fullch)
        def _():
            copy_one(CH * k, CH)
    if rem:
        @pl.when(s == 0)
        def _():
            copy_one(fullch * CH, rem)


def _make_sc_deg(nchunk, degn):
    @functools.partial(
        pl.kernel,
        out_type=jax.ShapeDtypeStruct((NC, degn), jnp.float32),
        mesh=_mesh(),
        scratch_types=[
            pltpu.VMEM((nchunk, CH), jnp.int32),
            pltpu.VMEM((nchunk, CH), jnp.float32),
            pltpu.VMEM((degn // NS,), jnp.float32),
            pltpu.VMEM_SHARED((degn,), jnp.float32),
        ],
        compiler_params=pltpu.CompilerParams(use_tc_tiling_on_sc=False),
    )
    def deg_kernel(dst_e, ew_e, out, dst_v, ew_v, zbuf, deg_sp):
        c = lax.axis_index("c")
        s = lax.axis_index("s")
        w = c * NS + s
        stripe = degn // NS

        def zb(i, carry):
            zbuf[pl.ds(16 * i, 16)] = jnp.zeros((16,), jnp.float32)
            return carry
        lax.fori_loop(0, stripe // 16, zb, 0)
        pltpu.sync_copy(zbuf, deg_sp.at[pl.ds(s * stripe, stripe)])
        plsc.subcore_barrier()

        pltpu.sync_copy(dst_e.at[w], dst_v)
        pltpu.sync_copy(ew_e.at[w], ew_v)

        def body(j, carry):
            pltpu.sync_copy(ew_v.at[j], deg_sp.at[dst_v.at[j]], add=True)
            return carry
        lax.fori_loop(0, nchunk, body, 0)

        plsc.subcore_barrier()
        pltpu.sync_copy(deg_sp.at[pl.ds(s * stripe, stripe)],
                        out.at[c, pl.ds(s * stripe, stripe)])

    return deg_kernel


def _make_sc_conv(n, d, nchunk, planes_per_core):
    P = planes_per_core
    NBG = 3
    NBS = 2
    assert nchunk % 6 == 0

    @functools.partial(
        pl.kernel,
        out_type=jax.ShapeDtypeStruct((NC * P, n, d), jnp.float32),
        mesh=_mesh(),
        scratch_types=[
            pltpu.VMEM((nchunk, CH), jnp.int32),
            pltpu.VMEM((nchunk, CH), jnp.int32),
            pltpu.VMEM((nchunk, CH), jnp.float32),
            [pltpu.VMEM((CH, d // 2), jnp.int32) for _ in range(NBG)],
            [pltpu.VMEM((CH, d), jnp.float32) for _ in range(NBS)],
            pltpu.VMEM_SHARED((n, d), jnp.float32),
            [pltpu.SemaphoreType.DMA for _ in range(NBG)],
            [pltpu.SemaphoreType.DMA for _ in range(NBS)],
        ],
        compiler_params=pltpu.CompilerParams(use_tc_tiling_on_sc=False),
    )
    def conv_kernel(table, src_e, dst_e, ew_e, out,
                    src_v, dst_v, ew_v, rowsb, rowsf, acc, gss, sss):
        c = lax.axis_index("c")
        s = lax.axis_index("s")
        w = s

        pltpu.sync_copy(src_e.at[w], src_v)
        pltpu.sync_copy(dst_e.at[w], dst_v)
        pltpu.sync_copy(ew_e.at[w], ew_v)

        for p in range(P):
            plane = c * P + p

            _zero_rows(rowsf[0], CH, d)

            def zero_copy(off, nr):
                pltpu.sync_copy(rowsf[0].at[pl.ds(0, nr)],
                                acc.at[pl.ds(off, nr)])
            _chunked_rows_copy(n, s, zero_copy)
            plsc.subcore_barrier()

            tbl = table.at[plane]

            def start_gather(bg, jj):
                pltpu.async_copy(tbl.at[src_v.at[jj]], rowsb[bg], gss[bg])

            def wait_gather(bg):
                pltpu.make_async_copy(
                    tbl.at[src_v.at[0]], rowsb[bg], gss[bg]).wait()

            def scale(bg, bs, jj):
                @plsc.parallel_loop(0, CH // 16, unroll=1)
                def _(g):
                    vec = ew_v[jj, pl.ds(16 * g, 16)]
                    for lane in range(16):
                        ewb = jnp.full((16,), vec[lane])
                        r = 16 * g + lane
                        for gg in range(d // 32):
                            wrd = rowsb[bg][r, pl.ds(16 * gg, 16)]
                            u0 = lax.bitcast_convert_type(
                                jnp.left_shift(wrd, 16), jnp.float32)
                            u1 = lax.bitcast_convert_type(
                                jnp.bitwise_and(wrd, jnp.int32(-65536)),
                                jnp.float32)
                            rowsf[bs][r, pl.ds(32 * gg, 16)] = u0 * ewb
                            rowsf[bs][r, pl.ds(32 * gg + 16, 16)] = u1 * ewb

            def start_scatter(bs, jj):
                pltpu.async_copy(rowsf[bs], acc.at[dst_v.at[jj]], sss[bs],
                                 add=True)

            def wait_scatter(bs):
                pltpu.make_async_copy(
                    rowsf[bs], acc.at[dst_v.at[0]], sss[bs]).wait()

            for bg in range(NBG):
                start_gather(bg, bg)

            def step(jj, bg, bs):
                @pl.when(jj >= NBS)
                def _():
                    wait_scatter(bs)
                wait_gather(bg)
                scale(bg, bs, jj)
                start_scatter(bs, jj)

                @pl.when(jj + NBG < nchunk)
                def _():
                    start_gather(bg, jj + NBG)

            def body(m, carry):
                for k in range(6):
                    step(6 * m + k, k % NBG, k % NBS)
                return carry
            lax.fori_loop(0, nchunk // 6, body, 0)
            for bs in range(NBS):
                wait_scatter(bs)

            plsc.subcore_barrier()

            def out_copy(off, nr):
                pltpu.sync_copy(acc.at[pl.ds(off, nr)],
                                out.at[plane, pl.ds(off, nr)])
            _chunked_rows_copy(n, s, out_copy)

    return conv_kernel


def _tc_k1a(x, wfc, bfc2, w1, n, blk):
    din, h1 = wfc.shape
    h2 = w1.shape[1]

    def body(x_ref, wfc_ref, bfc_ref, w1_ref, p1_ref):
        h0 = jnp.maximum(
            jnp.dot(x_ref[...], wfc_ref[...],
                    preferred_element_type=jnp.float32) + bfc_ref[...], 0.0)
        p1_ref[...] = jnp.dot(h0, w1_ref[...],
                              preferred_element_type=jnp.float32)

    return pl.pallas_call(
        body,
        grid=(n // blk,),
        in_specs=[
            pl.BlockSpec((blk, din), lambda i: (i, 0)),
            pl.BlockSpec((din, h1), lambda i: (0, 0)),
            pl.BlockSpec((1, h1), lambda i: (0, 0)),
            pl.BlockSpec((h1, h2), lambda i: (0, 0)),
        ],
        out_specs=pl.BlockSpec((blk, h2), lambda i: (i, 0)),
        out_shape=jax.ShapeDtypeStruct((n, h2), jnp.float32),
    )(x, wfc, bfc2, w1)


def _pack_bf16_pairs(x):
    lo = jnp.concatenate([x[:, 0:16], x[:, 32:48]], axis=1)
    hi = jnp.concatenate([x[:, 16:32], x[:, 48:64]], axis=1)

    def rne(v):
        b = lax.bitcast_convert_type(v, jnp.int32)
        r = (b + 0x7FFF + jnp.bitwise_and(lax.shift_right_logical(b, 16), 1))
        return jnp.bitwise_and(lax.shift_right_logical(r, 16), 0xFFFF)

    return jnp.bitwise_or(rne(lo), lax.shift_left(rne(hi), 16))


def _tc_k1b(p1, degpair, n, blk):
    h2 = p1.shape[1]

    def body(p1_ref, deg_ref, hp_ref, dinv_ref):
        deg = deg_ref[0, 0] + deg_ref[0, 1] + 1.0
        dinv = jnp.where(deg > 0, lax.rsqrt(jnp.maximum(deg, 1e-12)), 0.0)
        dinv_ref[...] = dinv[:, None]
        hp = jnp.concatenate(
            [p1_ref[...] * dinv[:, None],
             jnp.zeros((blk, 4 * 64 - h2), jnp.float32)], axis=1)
        for p in range(4):
            hp_ref[p] = _pack_bf16_pairs(hp[:, 64 * p:64 * (p + 1)])

    return pl.pallas_call(
        body,
        grid=(n // blk,),
        in_specs=[
            pl.BlockSpec((blk, h2), lambda i: (i, 0)),
            pl.BlockSpec((1, NC, blk), lambda i: (i, 0, 0)),
        ],
        out_specs=[
            pl.BlockSpec((4, blk, 32), lambda i: (0, i, 0)),
            pl.BlockSpec((blk, 1), lambda i: (i, 0)),
        ],
        out_shape=[
            jax.ShapeDtypeStruct((4, n, 32), jnp.int32),
            jax.ShapeDtypeStruct((n, 1), jnp.float32),
        ],
    )(p1, degpair)


def _tc_k2(acc1, p1, dinv, b1r, w2, n, blk):
    h2, dout = w2.shape

    def body(acc_ref, p1_ref, dinv_ref, b1_ref, w2_ref, hp2_ref, p2d_ref):
        acat = jnp.concatenate([acc_ref[p] for p in range(4)], axis=1)
        ssum = acat[:, :h2] + p1_ref[...] * dinv_ref[...]
        h1 = jnp.maximum(ssum * dinv_ref[...] + b1_ref[...], 0.0)
        p2 = jnp.dot(h1, w2_ref[...], preferred_element_type=jnp.float32)
        p2d = p2 * dinv_ref[...]
        p2d_ref[...] = p2d
        hp2_ref[0] = _pack_bf16_pairs(p2d[:, :dout // 2])
        hp2_ref[1] = _pack_bf16_pairs(p2d[:, dout // 2:])

    return pl.pallas_call(
        body,
        grid=(n // blk,),
        in_specs=[
            pl.BlockSpec((4, blk, 64), lambda i: (0, i, 0)),
            pl.BlockSpec((blk, h2), lambda i: (i, 0)),
            pl.BlockSpec((blk, 1), lambda i: (i, 0)),
            pl.BlockSpec((1, h2), lambda i: (0, 0)),
            pl.BlockSpec((h2, dout), lambda i: (0, 0)),
        ],
        out_specs=[
            pl.BlockSpec((NC, blk, dout // 4), lambda i: (0, i, 0)),
            pl.BlockSpec((blk, dout), lambda i: (i, 0)),
        ],
        out_shape=[
            jax.ShapeDtypeStruct((NC, n, dout // 4), jnp.int32),
            jax.ShapeDtypeStruct((n, dout), jnp.float32),
        ],
    )(acc1, p1, dinv, b1r, w2)


def _tc_k3(acc2, p2d, dinv, b2r, n, blk):
    dout = b2r.shape[1]

    def body(acc_ref, p2d_ref, dinv_ref, b2_ref, out_ref):
        ssum = jnp.concatenate([acc_ref[0], acc_ref[1]], axis=1) + p2d_ref[...]
        out_ref[...] = jnp.maximum(ssum * dinv_ref[...] + b2_ref[...], 0.0)

    return pl.pallas_call(
        body,
        grid=(n // blk,),
        in_specs=[
            pl.BlockSpec((NC, blk, dout // 2), lambda i: (0, i, 0)),
            pl.BlockSpec((blk, dout), lambda i: (i, 0)),
            pl.BlockSpec((blk, 1), lambda i: (i, 0)),
            pl.BlockSpec((1, dout), lambda i: (0, 0)),
        ],
        out_specs=pl.BlockSpec((blk, dout), lambda i: (i, 0)),
        out_shape=jax.ShapeDtypeStruct((n, dout), jnp.float32),
    )(acc2, p2d, dinv, b2r)


def kernel(x, edge_idx, edge_attr, Wfc, bfc, W1, b1, W2, b2):
    n, _ = x.shape
    e = edge_attr.shape[0]
    dout = W2.shape[1]
    blk = 2000

    nchunk = -(-e // (NW * CH))
    epad = NW * nchunk * CH
    nchunk1 = -(-e // (NS * CH))
    nchunk1 = -(-nchunk1 // 6) * 6
    epad1 = NS * nchunk1 * CH
    degn = -(-n // (NS * 16)) * (NS * 16)

    dst = jnp.pad(edge_idx[1], (0, epad - e)).reshape(NW, nchunk, CH)
    ew = jnp.pad(edge_attr, (0, epad - e)).reshape(NW, nchunk, CH)
    src1 = jnp.pad(edge_idx[0], (0, epad1 - e)).reshape(NS, nchunk1, CH)
    dst1 = jnp.pad(edge_idx[1], (0, epad1 - e)).reshape(NS, nchunk1, CH)
    ew1 = jnp.pad(edge_attr, (0, epad1 - e)).reshape(NS, nchunk1, CH)

    p1 = _tc_k1a(x, Wfc, bfc.reshape(1, -1), W1, n, blk)
    degpair = _make_sc_deg(nchunk, degn)(dst, ew)
    degblk = degpair[:, :n].reshape(NC, n // blk, blk).transpose(1, 0, 2)
    hp1, dinv = _tc_k1b(p1, degblk, n, blk)

    acc1 = _make_sc_conv(n, 64, nchunk1, 2)(hp1, src1, dst1, ew1)

    hp2, p2d = _tc_k2(acc1, p1, dinv, b1.reshape(1, -1), W2, n, blk)

    acc2 = _make_sc_conv(n, dout // 2, nchunk1, 1)(hp2, src1, dst1, ew1)

    return _tc_k3(acc2, p2d, dinv, b2.reshape(1, -1), n, blk)
